# baseline (speedup 1.0000x reference)
import jax
import jax.numpy as jnp
from jax import lax
from jax.experimental import pallas as pl
from jax.experimental.pallas import tpu as pltpu
from jax.experimental.pallas import tpu_sc as plsc

NUM_K = 8192
DIM = 256
ROWS = 256
NROWS = 8192
NWORK = 32
RPW = NROWS // NWORK


def _normalize_body(x_ref, o_ref):
    x = x_ref[...]
    n = jnp.sqrt(jnp.sum(x * x, axis=-1, keepdims=True))
    o_ref[...] = x / jnp.maximum(n, 1e-6)


def _vq_body(h_ref, cbn_ref, q_ref, idx_ref):
    hb = h_ref[...]
    n = jnp.sqrt(jnp.sum(hb * hb, axis=-1, keepdims=True))
    hn = hb / jnp.maximum(n, 1e-6)
    cbn = cbn_ref[...]
    sims = lax.dot_general(hn, cbn, (((1,), (1,)), ((), ())),
                           preferred_element_type=jnp.float32)
    q_ref[...] = jnp.where(sims > 3.0, 1.0, 0.0)
    idx_ref[0, 0, :] = jnp.zeros((ROWS,), jnp.int32)


def _gather_body(cbn_hbm, idx_hbm, out_hbm, idx_v, rows_v, sem):
    wid = lax.axis_index("s") * 2 + lax.axis_index("c")
    base = wid * RPW
    pltpu.sync_copy(idx_hbm.at[pl.ds(base, RPW)], idx_v)
    pltpu.async_copy(cbn_hbm.at[idx_v], rows_v, sem).wait()
    pltpu.sync_copy(rows_v, out_hbm.at[pl.ds(base, RPW)])


def _normalize_cb(codebook):
    return pl.pallas_call(
        _normalize_body,
        grid=(8,),
        in_specs=[pl.BlockSpec((NUM_K // 8, DIM), lambda i: (i, 0))],
        out_specs=pl.BlockSpec((NUM_K // 8, DIM), lambda i: (i, 0)),
        out_shape=jax.ShapeDtypeStruct((NUM_K, DIM), jnp.float32),
    )(codebook)


def _vq(h_flat, cbn):
    return pl.pallas_call(
        _vq_body,
        grid=(NROWS // ROWS,),
        in_specs=[
            pl.BlockSpec((ROWS, DIM), lambda i: (i, 0)),
            pl.BlockSpec((NUM_K, DIM), lambda i: (0, 0)),
        ],
        out_specs=[
            pl.BlockSpec((ROWS, NUM_K), lambda i: (i, 0)),
            pl.BlockSpec((1, 1, ROWS), lambda i: (i, 0, 0)),
        ],
        out_shape=[
            jax.ShapeDtypeStruct((NROWS, NUM_K), jnp.float32),
            jax.ShapeDtypeStruct((NROWS // ROWS, 1, ROWS), jnp.int32),
        ],
    )(h_flat, cbn)


def _sc_gather(cbn, idx_flat):
    mesh = plsc.VectorSubcoreMesh(core_axis_name="c", subcore_axis_name="s")
    f = pl.kernel(
        _gather_body,
        mesh=mesh,
        out_type=jax.ShapeDtypeStruct((NROWS, DIM), jnp.float32),
        scratch_types=[
            pltpu.VMEM((RPW,), jnp.int32),
            pltpu.VMEM((RPW, DIM), jnp.float32),
            pltpu.SemaphoreType.DMA,
        ],
    )
    return f(cbn, idx_flat)


def kernel(h, codebook):
    B, S, D = h.shape
    h_flat = h.reshape(-1, D)
    cbn = _normalize_cb(codebook)
    q_flat, idx3 = _vq(h_flat, cbn)
    idx_flat = idx3.reshape(NROWS)
    c_flat = _sc_gather(cbn, idx_flat)
    q = q_flat.reshape(B, S, NUM_K)
    c = c_flat.reshape(B, S, D)
    indices = idx3.reshape(B, S)
    return (q, c, c, c, indices)


# speedup vs baseline: 2.5632x; 2.5632x over previous
import jax
import jax.numpy as jnp
from jax import lax
from jax.experimental import pallas as pl
from jax.experimental.pallas import tpu as pltpu
from jax.experimental.pallas import tpu_sc as plsc

NUM_K = 8192
DIM = 256
ROWS = 256
NROWS = 8192
NWORK = 32
RPW = NROWS // NWORK


def _normalize_body(x_ref, o_ref):
    x = x_ref[...]
    n = jnp.sqrt(jnp.sum(x * x, axis=-1, keepdims=True))
    o_ref[...] = x / jnp.maximum(n, 1e-6)


def _vq_body(h_ref, cbn_ref, q_ref, idx_ref):
    hb = h_ref[...]
    n = jnp.sqrt(jnp.sum(hb * hb, axis=-1, keepdims=True))
    hn = hb / jnp.maximum(n, 1e-6)
    cbn = cbn_ref[...]
    sims = lax.dot_general(hn, cbn, (((1,), (1,)), ((), ())),
                           preferred_element_type=jnp.float32)
    smax = jnp.max(sims, axis=1, keepdims=True)
    m = 2.0 - 2.0 * smax
    d = 2.0 - 2.0 * sims
    iota_f = lax.broadcasted_iota(jnp.int32, d.shape, 1).astype(jnp.float32)
    cand = jnp.where(d == m, iota_f, float(NUM_K))
    idx_f = jnp.min(cand, axis=1)
    q_ref[...] = jnp.where(cand == idx_f[:, None], 1.0, 0.0)
    idx_ref[0, 0, :] = idx_f.astype(jnp.int32)


def _gather_body(cbn_hbm, idx_hbm, out_hbm, idx_v, rows_v, sem):
    wid = lax.axis_index("s") * 2 + lax.axis_index("c")
    base = wid * RPW
    pltpu.sync_copy(idx_hbm.at[pl.ds(base, RPW)], idx_v)
    pltpu.async_copy(cbn_hbm.at[idx_v], rows_v, sem).wait()
    pltpu.sync_copy(rows_v, out_hbm.at[pl.ds(base, RPW)])


def _normalize_cb(codebook):
    return pl.pallas_call(
        _normalize_body,
        grid=(8,),
        in_specs=[pl.BlockSpec((NUM_K // 8, DIM), lambda i: (i, 0))],
        out_specs=pl.BlockSpec((NUM_K // 8, DIM), lambda i: (i, 0)),
        out_shape=jax.ShapeDtypeStruct((NUM_K, DIM), jnp.float32),
    )(codebook)


def _vq(h_flat, cbn):
    return pl.pallas_call(
        _vq_body,
        grid=(NROWS // ROWS,),
        in_specs=[
            pl.BlockSpec((ROWS, DIM), lambda i: (i, 0)),
            pl.BlockSpec((NUM_K, DIM), lambda i: (0, 0)),
        ],
        out_specs=[
            pl.BlockSpec((ROWS, NUM_K), lambda i: (i, 0)),
            pl.BlockSpec((1, 1, ROWS), lambda i: (i, 0, 0)),
        ],
        out_shape=[
            jax.ShapeDtypeStruct((NROWS, NUM_K), jnp.float32),
            jax.ShapeDtypeStruct((NROWS // ROWS, 1, ROWS), jnp.int32),
        ],
    )(h_flat, cbn)


def _sc_gather(cbn, idx_flat):
    mesh = plsc.VectorSubcoreMesh(core_axis_name="c", subcore_axis_name="s")
    f = pl.kernel(
        _gather_body,
        mesh=mesh,
        out_type=jax.ShapeDtypeStruct((NROWS, DIM), jnp.float32),
        scratch_types=[
            pltpu.VMEM((RPW,), jnp.int32),
            pltpu.VMEM((RPW, DIM), jnp.float32),
            pltpu.SemaphoreType.DMA,
        ],
    )
    return f(cbn, idx_flat)


def kernel(h, codebook):
    B, S, D = h.shape
    h_flat = h.reshape(-1, D)
    cbn = _normalize_cb(codebook)
    q_flat, idx3 = _vq(h_flat, cbn)
    idx_flat = idx3.reshape(NROWS)
    c_flat = _sc_gather(cbn, idx_flat)
    q = q_flat.reshape(B, S, NUM_K)
    c = c_flat.reshape(B, S, D)
    indices = idx3.reshape(B, S)
    return (q, c, c, c, indices)


# speedup vs baseline: 2.6263x; 1.0246x over previous
import jax
import jax.numpy as jnp
from jax import lax
from jax.experimental import pallas as pl
from jax.experimental.pallas import tpu as pltpu
from jax.experimental.pallas import tpu_sc as plsc

NUM_K = 8192
DIM = 256
ROWS = 256
NROWS = 8192
NWORK = 32
RPW = NROWS // NWORK


def _normalize_body(x_ref, o_ref):
    x = x_ref[...]
    n = jnp.sqrt(jnp.sum(x * x, axis=-1, keepdims=True))
    o_ref[...] = x / jnp.maximum(n, 1e-6)


def _vq_body(h_ref, cbn_ref, q_ref, idx_ref):
    hb = h_ref[...]
    n = jnp.sqrt(jnp.sum(hb * hb, axis=-1, keepdims=True))
    hn = hb / jnp.maximum(n, 1e-6)
    cbn = cbn_ref[...]
    sims = lax.dot_general(hn, cbn, (((1,), (1,)), ((), ())),
                           preferred_element_type=jnp.float32)
    d = 2.0 - 2.0 * sims
    m = jnp.min(d, axis=1, keepdims=True)
    iota_f = lax.broadcasted_iota(jnp.int32, d.shape, 1).astype(jnp.float32)
    cand = jnp.where(d == m, iota_f, float(NUM_K))
    idx_f = jnp.min(cand, axis=1)
    q_ref[...] = jnp.where(cand == idx_f[:, None], 1.0, 0.0)
    idx_ref[0, 0, :] = idx_f.astype(jnp.int32)


def _gather_body(cbn_hbm, idx_hbm, out_hbm, idx_v, rows_v, sem):
    wid = lax.axis_index("s") * 2 + lax.axis_index("c")
    base = wid * RPW
    pltpu.sync_copy(idx_hbm.at[pl.ds(base, RPW)], idx_v)
    pltpu.async_copy(cbn_hbm.at[idx_v], rows_v, sem).wait()
    pltpu.sync_copy(rows_v, out_hbm.at[pl.ds(base, RPW)])


def _normalize_cb(codebook):
    return pl.pallas_call(
        _normalize_body,
        grid=(8,),
        in_specs=[pl.BlockSpec((NUM_K // 8, DIM), lambda i: (i, 0))],
        out_specs=pl.BlockSpec((NUM_K // 8, DIM), lambda i: (i, 0)),
        out_shape=jax.ShapeDtypeStruct((NUM_K, DIM), jnp.float32),
    )(codebook)


def _vq(h_flat, cbn):
    return pl.pallas_call(
        _vq_body,
        grid=(NROWS // ROWS,),
        in_specs=[
            pl.BlockSpec((ROWS, DIM), lambda i: (i, 0)),
            pl.BlockSpec((NUM_K, DIM), lambda i: (0, 0)),
        ],
        out_specs=[
            pl.BlockSpec((ROWS, NUM_K), lambda i: (i, 0)),
            pl.BlockSpec((1, 1, ROWS), lambda i: (i, 0, 0)),
        ],
        out_shape=[
            jax.ShapeDtypeStruct((NROWS, NUM_K), jnp.float32),
            jax.ShapeDtypeStruct((NROWS // ROWS, 1, ROWS), jnp.int32),
        ],
    )(h_flat, cbn)


def _sc_gather(cbn, idx_flat):
    mesh = plsc.VectorSubcoreMesh(core_axis_name="c", subcore_axis_name="s")
    f = pl.kernel(
        _gather_body,
        mesh=mesh,
        out_type=jax.ShapeDtypeStruct((NROWS, DIM), jnp.float32),
        scratch_types=[
            pltpu.VMEM((RPW,), jnp.int32),
            pltpu.VMEM((RPW, DIM), jnp.float32),
            pltpu.SemaphoreType.DMA,
        ],
    )
    return f(cbn, idx_flat)


def kernel(h, codebook):
    B, S, D = h.shape
    h_flat = h.reshape(-1, D)
    cbn = _normalize_cb(codebook)
    q_flat, idx3 = _vq(h_flat, cbn)
    idx_flat = idx3.reshape(NROWS)
    c_flat = _sc_gather(cbn, idx_flat)
    q = q_flat.reshape(B, S, NUM_K)
    c = c_flat.reshape(B, S, D)
    indices = idx3.reshape(B, S)
    return (q, c, c, c, indices)


# speedup vs baseline: 2.7125x; 1.0328x over previous
import jax
import jax.numpy as jnp
from jax import lax
from jax.experimental import pallas as pl
from jax.experimental.pallas import tpu as pltpu
from jax.experimental.pallas import tpu_sc as plsc

NUM_K = 8192
DIM = 256
ROWS = 256
NROWS = 8192
NWORK = 32
RPW = NROWS // NWORK


def _vq_body(h_ref, cb_ref, q_ref, idx_ref, cbn_ref):
    @pl.when(pl.program_id(0) == 0)
    def _():
        x = cb_ref[...]
        cn = jnp.sqrt(jnp.sum(x * x, axis=-1, keepdims=True))
        cbn_ref[...] = x / jnp.maximum(cn, 1e-6)

    hb = h_ref[...]
    n = jnp.sqrt(jnp.sum(hb * hb, axis=-1, keepdims=True))
    hn = hb / jnp.maximum(n, 1e-6)
    cbn = cbn_ref[...]
    sims = lax.dot_general(hn, cbn, (((1,), (1,)), ((), ())),
                           preferred_element_type=jnp.float32)
    d = 2.0 - 2.0 * sims
    m = jnp.min(d, axis=1, keepdims=True)
    iota_f = lax.broadcasted_iota(jnp.int32, d.shape, 1).astype(jnp.float32)
    cand = jnp.where(d == m, iota_f, float(NUM_K))
    idx_f = jnp.min(cand, axis=1)
    q_ref[...] = jnp.where(cand == idx_f[:, None], 1.0, 0.0)
    idx_ref[0, 0, :] = idx_f.astype(jnp.int32)


def _gather_body(cbn_hbm, idx_hbm, out_hbm, idx_v, rows_v, sem):
    wid = lax.axis_index("s") * 2 + lax.axis_index("c")
    base = wid * RPW
    pltpu.sync_copy(idx_hbm.at[pl.ds(base, RPW)], idx_v)
    pltpu.async_copy(cbn_hbm.at[idx_v], rows_v, sem).wait()
    pltpu.sync_copy(rows_v, out_hbm.at[pl.ds(base, RPW)])


def _vq(h_flat, codebook):
    return pl.pallas_call(
        _vq_body,
        grid=(NROWS // ROWS,),
        in_specs=[
            pl.BlockSpec((ROWS, DIM), lambda i: (i, 0)),
            pl.BlockSpec((NUM_K, DIM), lambda i: (0, 0)),
        ],
        out_specs=[
            pl.BlockSpec((ROWS, NUM_K), lambda i: (i, 0)),
            pl.BlockSpec((1, 1, ROWS), lambda i: (i, 0, 0)),
            pl.BlockSpec((NUM_K, DIM), lambda i: (0, 0)),
        ],
        out_shape=[
            jax.ShapeDtypeStruct((NROWS, NUM_K), jnp.float32),
            jax.ShapeDtypeStruct((NROWS // ROWS, 1, ROWS), jnp.int32),
            jax.ShapeDtypeStruct((NUM_K, DIM), jnp.float32),
        ],
    )(h_flat, codebook)


def _sc_gather(cbn, idx_flat):
    mesh = plsc.VectorSubcoreMesh(core_axis_name="c", subcore_axis_name="s")
    f = pl.kernel(
        _gather_body,
        mesh=mesh,
        out_type=jax.ShapeDtypeStruct((NROWS, DIM), jnp.float32),
        scratch_types=[
            pltpu.VMEM((RPW,), jnp.int32),
            pltpu.VMEM((RPW, DIM), jnp.float32),
            pltpu.SemaphoreType.DMA,
        ],
    )
    return f(cbn, idx_flat)


def kernel(h, codebook):
    B, S, D = h.shape
    h_flat = h.reshape(-1, D)
    q_flat, idx3, cbn = _vq(h_flat, codebook)
    idx_flat = idx3.reshape(NROWS)
    c_flat = _sc_gather(cbn, idx_flat)
    q = q_flat.reshape(B, S, NUM_K)
    c = c_flat.reshape(B, S, D)
    indices = idx3.reshape(B, S)
    return (q, c, c, c, indices)
